# Initial kernel scaffold; baseline (speedup 1.0000x reference)
import functools

import jax
import jax.numpy as jnp
from jax import lax
from jax.experimental import pallas as pl
from jax.experimental.pallas import tpu as pltpu
from jax.experimental.pallas import tpu_sc as plsc

N = 10000
D = 128
NT = 10240
NW = 32
CH = 128
NCHUNK = 81
EPT = NCHUNK * CH
E_PAD = NW * EPT
ROWS_PER_TILE = NT // 16
ZCH = 128
WB_STEPS = ROWS_PER_TILE // ZCH


BLK = 256
GRID = NT // BLK


def _pre_body(x_ref, wl_ref, bl_ref, wm_ref, bm_ref, o_ref):
    h = jnp.maximum(
        jnp.dot(x_ref[...], wl_ref[...], preferred_element_type=jnp.float32)
        + bl_ref[...], 0.0)
    v = jnp.dot(h, wm_ref[...], preferred_element_type=jnp.float32) + bm_ref[...]
    rows = pl.program_id(0) * BLK + lax.broadcasted_iota(jnp.int32, (BLK, 1), 0)
    o_ref[...] = jnp.where(rows < N, v, 0.0)


def _tc_pre(x_pad, Wl, bl, Wmh, bm):
    return pl.pallas_call(
        _pre_body,
        grid=(GRID,),
        in_specs=[
            pl.BlockSpec((BLK, D), lambda i: (i, 0)),
            pl.BlockSpec((D, D), lambda i: (0, 0)),
            pl.BlockSpec((1, D), lambda i: (0, 0)),
            pl.BlockSpec((D, D), lambda i: (0, 0)),
            pl.BlockSpec((1, D), lambda i: (0, 0)),
        ],
        out_specs=pl.BlockSpec((BLK, D), lambda i: (i, 0)),
        out_shape=jax.ShapeDtypeStruct((NT, D), jnp.float32),
    )(x_pad, Wl, bl.reshape(1, D), Wmh, bm.reshape(1, D))


def _out_body(a_ref, c_ref, x_ref, wll_ref, bll_ref, wr_ref, o_ref):
    cnt = c_ref[0, :, 0:1] + c_ref[1, :, 0:1]
    agg = (a_ref[0] + a_ref[1]) * (1.0 / jnp.maximum(cnt, 1.0))
    v = (jnp.dot(agg, wll_ref[...], preferred_element_type=jnp.float32)
         + bll_ref[...]
         + jnp.dot(x_ref[...], wr_ref[...], preferred_element_type=jnp.float32))
    rows = pl.program_id(0) * BLK + lax.broadcasted_iota(jnp.int32, (BLK, 1), 0)
    o_ref[...] = jnp.where(rows < N, jnp.maximum(v, 0.0), 0.0)


def _tc_out(aggp, cntp, x_pad, Wll, bll, Wr):
    return pl.pallas_call(
        _out_body,
        grid=(GRID,),
        in_specs=[
            pl.BlockSpec((2, BLK, D), lambda i: (0, i, 0)),
            pl.BlockSpec((2, BLK, 16), lambda i: (0, i, 0)),
            pl.BlockSpec((BLK, D), lambda i: (i, 0)),
            pl.BlockSpec((D, D), lambda i: (0, 0)),
            pl.BlockSpec((1, D), lambda i: (0, 0)),
            pl.BlockSpec((D, D), lambda i: (0, 0)),
        ],
        out_specs=pl.BlockSpec((BLK, D), lambda i: (i, 0)),
        out_shape=jax.ShapeDtypeStruct((NT, D), jnp.float32),
    )(aggp, cntp, x_pad, Wll, bll.reshape(1, D), Wr)


def _fin_body(x_ref, wd_ref, bd_ref, wf_ref, bf_ref, o_ref):
    h = jnp.maximum(
        jnp.dot(x_ref[...], wd_ref[...], preferred_element_type=jnp.float32)
        + bd_ref[...], 0.0)
    o_ref[...] = (jnp.dot(h, wf_ref[...], preferred_element_type=jnp.float32)
                  + bf_ref[...])


def _tc_fin(h, Wd, bd, Wf, bf):
    return pl.pallas_call(
        _fin_body,
        grid=(GRID,),
        in_specs=[
            pl.BlockSpec((BLK, D), lambda i: (i, 0)),
            pl.BlockSpec((D, D), lambda i: (0, 0)),
            pl.BlockSpec((1, D), lambda i: (0, 0)),
            pl.BlockSpec((D, D), lambda i: (0, 0)),
            pl.BlockSpec((1, D), lambda i: (0, 0)),
        ],
        out_specs=pl.BlockSpec((BLK, D), lambda i: (i, 0)),
        out_shape=jax.ShapeDtypeStruct((NT, D), jnp.float32),
    )(h, Wd, bd.reshape(1, D), Wf, bf.reshape(1, D))


def _make_sc_kernel(with_counts):
    out_type = [jax.ShapeDtypeStruct((2, NT, D), jnp.float32)]
    if with_counts:
        out_type.append(jax.ShapeDtypeStruct((2, NT, 16), jnp.float32))

    scratch = [
        pltpu.VMEM((NCHUNK, CH), jnp.int32),
        pltpu.VMEM((NCHUNK, CH), jnp.int32),
        pltpu.VMEM((NCHUNK, CH), jnp.float32),
        pltpu.VMEM((D,), jnp.float32),
        pltpu.VMEM((3, CH, D), jnp.float32),
        pltpu.VMEM((CH, 16), jnp.float32),
        pltpu.VMEM((ZCH, D), jnp.float32),
        pltpu.VMEM((ZCH, 16), jnp.float32),
        pltpu.VMEM_SHARED((NT, D), jnp.float32),
        pltpu.VMEM_SHARED((NT, 16), jnp.float32),
        pltpu.SemaphoreType.DMA,
        pltpu.SemaphoreType.DMA,
        pltpu.SemaphoreType.DMA,
        pltpu.SemaphoreType.DMA,
        pltpu.SemaphoreType.DMA,
        pltpu.SemaphoreType.DMA,
    ]

    mesh = plsc.VectorSubcoreMesh(core_axis_name="c", subcore_axis_name="s")

    @functools.partial(pl.kernel, out_type=out_type, mesh=mesh,
                       scratch_types=scratch)
    def sc_kernel(hm_hbm, src_hbm, dst_hbm, ea_hbm, wm_hbm, *refs):
        if with_counts:
            agg_out, cnt_out = refs[0], refs[1]
            rest = refs[2:]
        else:
            agg_out = refs[0]
            cnt_out = None
            rest = refs[1:]
        (src_v, dst_v, ea_v, wm_v, rows3, ones_v, zb, zbc,
         aggS, cntS, g0, g1, g2, s0, s1, s2) = rest
        gsem = (g0, g1, g2)
        ssem = (s0, s1, s2)

        c = lax.axis_index("c")
        s = lax.axis_index("s")
        w = c * 16 + s

        base_row = w * NCHUNK
        pltpu.sync_copy(src_hbm.at[pl.ds(base_row, NCHUNK), :], src_v)
        pltpu.sync_copy(dst_hbm.at[pl.ds(base_row, NCHUNK), :], dst_v)
        pltpu.sync_copy(ea_hbm.at[pl.ds(base_row, NCHUNK), :], ea_v)
        pltpu.sync_copy(wm_hbm, wm_v)

        def _fill_row(r, _):
            for j in range(D // 16):
                zb[r, pl.ds(j * 16, 16)] = jnp.zeros((16,), jnp.float32)
            zbc[r, :] = jnp.zeros((16,), jnp.float32)
            ones_v[r, :] = jnp.ones((16,), jnp.float32)
            return 0
        lax.fori_loop(0, ZCH, _fill_row, 0)

        for k in range(WB_STEPS):
            r0 = (s * WB_STEPS + k) * ZCH
            pltpu.sync_copy(zb, aggS.at[pl.ds(r0, ZCH), :])
            if with_counts:
                pltpu.sync_copy(zbc, cntS.at[pl.ds(r0, ZCH), :])
        plsc.subcore_barrier()

        def start_gather(b, g):
            pltpu.async_copy(hm_hbm.at[src_v.at[g]], rows3.at[b], gsem[b])

        def wait_gather(b):
            pltpu.make_async_copy(hm_hbm.at[pl.ds(0, CH), :], rows3.at[b],
                                  gsem[b]).wait()

        def start_scatter(b, g):
            pltpu.async_copy(rows3.at[b], aggS.at[dst_v.at[g]], ssem[b],
                             add=True)
            if with_counts:
                pltpu.async_copy(ones_v, cntS.at[dst_v.at[g]], ssem[b],
                                 add=True)

        def wait_scatter(b):
            pltpu.make_async_copy(hm_hbm.at[pl.ds(0, CH), :], rows3.at[b],
                                  ssem[b]).wait()
            if with_counts:
                pltpu.make_async_copy(hm_hbm.at[pl.ds(0, CH), pl.ds(0, 16)],
                                      ones_v, ssem[b]).wait()

        def compute_chunk(b, g):
            wvs = tuple(wm_v[pl.ds(16 * j, 16)] for j in range(D // 16))

            def ebody(e, wv):
                gi = jnp.full((16,), g, jnp.int32)
                ei = jnp.full((16,), e, jnp.int32)
                eab = plsc.load_gather(ea_v, [gi, ei])
                for j in range(D // 16):
                    sl = pl.ds(j * 16, 16)
                    v = rows3[b, e, sl]
                    rows3[b, e, sl] = jnp.maximum(v + eab * wv[j], 0.0)
                return wv
            lax.fori_loop(0, CH, ebody, wvs)

        start_gather(0, 0)
        start_gather(1, 1)

        wait_gather(0); compute_chunk(0, 0); start_scatter(0, 0)
        start_gather(2, 2)
        wait_gather(1); compute_chunk(1, 1); start_scatter(1, 1)
        wait_scatter(0); start_gather(0, 3)
        wait_gather(2); compute_chunk(2, 2); start_scatter(2, 2)
        wait_scatter(1); start_gather(1, 4)

        def triple(t, _):
            for i in range(3):
                ch = 3 * t + 3 + i
                wait_gather(i)
                compute_chunk(i, ch)
                start_scatter(i, ch)
                wait_scatter((i + 2) % 3)
                start_gather((i + 2) % 3, ch + 2)
            return 0
        lax.fori_loop(0, (NCHUNK - 6) // 3, triple, 0)

        cA, cB, cC = NCHUNK - 3, NCHUNK - 2, NCHUNK - 1
        wait_gather(0); compute_chunk(0, cA); start_scatter(0, cA)
        wait_scatter(2); start_gather(2, cC)
        wait_gather(1); compute_chunk(1, cB); start_scatter(1, cB)
        wait_scatter(0)
        wait_gather(2); compute_chunk(2, cC); start_scatter(2, cC)
        wait_scatter(1)
        wait_scatter(2)

        plsc.subcore_barrier()

        for k in range(WB_STEPS):
            r0 = (s * WB_STEPS + k) * ZCH
            pltpu.sync_copy(aggS.at[pl.ds(r0, ZCH), :], zb)
            pltpu.sync_copy(zb, agg_out.at[c, pl.ds(r0, ZCH), :])
            if with_counts:
                pltpu.sync_copy(cntS.at[pl.ds(r0, ZCH), :], ones_v)
                pltpu.sync_copy(ones_v, cnt_out.at[c, pl.ds(r0, ZCH), :])

    return sc_kernel


_sc_edge0 = _make_sc_kernel(True)
_sc_edge1 = _make_sc_kernel(False)


def kernel(x, edge_index, edge_attr, W_lin_0, b_lin_0, W_msg_0, b_msg_0,
           W_l_0, b_l_0, W_r_0, W_lin_1, b_lin_1, W_msg_1, b_msg_1, W_l_1,
           b_l_1, W_r_1, W_dec, b_dec, W_fin, b_fin):
    x_pad = jnp.pad(x, ((0, NT - N), (0, 0)))
    src = edge_index[0].astype(jnp.int32)
    dst = edge_index[1].astype(jnp.int32)
    ea = edge_attr[:, 0].astype(jnp.float32)
    npad = E_PAD - src.shape[0]
    padidx = (jnp.arange(npad, dtype=jnp.int32) % (NT - N)) + N
    src_p = jnp.concatenate([src, padidx]).reshape(E_PAD // CH, CH)
    dst_p = jnp.concatenate([dst, padidx]).reshape(E_PAD // CH, CH)
    ea_p = jnp.concatenate([ea, jnp.zeros((npad,), jnp.float32)]
                           ).reshape(E_PAD // CH, CH)

    hm0 = _tc_pre(x_pad, W_lin_0, b_lin_0, W_msg_0[:D], b_msg_0)
    aggp0, cntp = _sc_edge0(hm0, src_p, dst_p, ea_p, W_msg_0[D])
    out0 = _tc_out(aggp0, cntp, x_pad, W_l_0, b_l_0, W_r_0)

    hm1 = _tc_pre(out0, W_lin_1, b_lin_1, W_msg_1[:D], b_msg_1)
    aggp1 = _sc_edge1(hm1, src_p, dst_p, ea_p, W_msg_1[D])
    out1 = _tc_out(aggp1, cntp, out0, W_l_1, b_l_1, W_r_1)

    logits = _tc_fin(out1, W_dec, b_dec, W_fin, b_fin)
    return logits[:N]


# speedup vs baseline: 3.2284x; 3.2284x over previous
import functools

import jax
import jax.numpy as jnp
from jax import lax
from jax.experimental import pallas as pl
from jax.experimental.pallas import tpu as pltpu
from jax.experimental.pallas import tpu_sc as plsc

N = 10000
D = 128
NT = 10240
NW = 32
CH = 64
SEG = 16
NSEG = 10
NCHUNK = SEG * NSEG
EPT = NCHUNK * CH
E_PAD = NW * EPT
ROWS_PER_TILE = NT // 16


BLK = 256
GRID = NT // BLK


def _pre_body(x_ref, wl_ref, bl_ref, wm_ref, bm_ref, o_ref):
    h = jnp.maximum(
        jnp.dot(x_ref[...], wl_ref[...], preferred_element_type=jnp.float32)
        + bl_ref[...], 0.0)
    v = jnp.dot(h, wm_ref[...], preferred_element_type=jnp.float32) + bm_ref[...]
    rows = pl.program_id(0) * BLK + lax.broadcasted_iota(jnp.int32, (BLK, 1), 0)
    o_ref[...] = jnp.where(rows < N, v, 0.0)


def _tc_pre(x_pad, Wl, bl, Wmh, bm):
    return pl.pallas_call(
        _pre_body,
        grid=(GRID,),
        in_specs=[
            pl.BlockSpec((BLK, D), lambda i: (i, 0)),
            pl.BlockSpec((D, D), lambda i: (0, 0)),
            pl.BlockSpec((1, D), lambda i: (0, 0)),
            pl.BlockSpec((D, D), lambda i: (0, 0)),
            pl.BlockSpec((1, D), lambda i: (0, 0)),
        ],
        out_specs=pl.BlockSpec((BLK, D), lambda i: (i, 0)),
        out_shape=jax.ShapeDtypeStruct((NT, D), jnp.float32),
    )(x_pad, Wl, bl.reshape(1, D), Wmh, bm.reshape(1, D))


def _out_body(a_ref, c_ref, x_ref, wll_ref, bll_ref, wr_ref, o_ref):
    cnt = c_ref[0, :, 0:1] + c_ref[1, :, 0:1]
    agg = (a_ref[0] + a_ref[1]) * (1.0 / jnp.maximum(cnt, 1.0))
    v = (jnp.dot(agg, wll_ref[...], preferred_element_type=jnp.float32)
         + bll_ref[...]
         + jnp.dot(x_ref[...], wr_ref[...], preferred_element_type=jnp.float32))
    rows = pl.program_id(0) * BLK + lax.broadcasted_iota(jnp.int32, (BLK, 1), 0)
    o_ref[...] = jnp.where(rows < N, jnp.maximum(v, 0.0), 0.0)


def _tc_out(aggp, cntp, x_pad, Wll, bll, Wr):
    return pl.pallas_call(
        _out_body,
        grid=(GRID,),
        in_specs=[
            pl.BlockSpec((2, BLK, D), lambda i: (0, i, 0)),
            pl.BlockSpec((2, BLK, D), lambda i: (0, i, 0)),
            pl.BlockSpec((BLK, D), lambda i: (i, 0)),
            pl.BlockSpec((D, D), lambda i: (0, 0)),
            pl.BlockSpec((1, D), lambda i: (0, 0)),
            pl.BlockSpec((D, D), lambda i: (0, 0)),
        ],
        out_specs=pl.BlockSpec((BLK, D), lambda i: (i, 0)),
        out_shape=jax.ShapeDtypeStruct((NT, D), jnp.float32),
    )(aggp, cntp, x_pad, Wll, bll.reshape(1, D), Wr)


def _fin_body(x_ref, wd_ref, bd_ref, wf_ref, bf_ref, o_ref):
    h = jnp.maximum(
        jnp.dot(x_ref[...], wd_ref[...], preferred_element_type=jnp.float32)
        + bd_ref[...], 0.0)
    o_ref[...] = (jnp.dot(h, wf_ref[...], preferred_element_type=jnp.float32)
                  + bf_ref[...])


def _tc_fin(h, Wd, bd, Wf, bf):
    return pl.pallas_call(
        _fin_body,
        grid=(GRID,),
        in_specs=[
            pl.BlockSpec((BLK, D), lambda i: (i, 0)),
            pl.BlockSpec((D, D), lambda i: (0, 0)),
            pl.BlockSpec((1, D), lambda i: (0, 0)),
            pl.BlockSpec((D, D), lambda i: (0, 0)),
            pl.BlockSpec((1, D), lambda i: (0, 0)),
        ],
        out_specs=pl.BlockSpec((BLK, D), lambda i: (i, 0)),
        out_shape=jax.ShapeDtypeStruct((NT, D), jnp.float32),
    )(h, Wd, bd.reshape(1, D), Wf, bf.reshape(1, D))


PK_MASK = 16383


def _sc_mesh():
    return plsc.VectorSubcoreMesh(core_axis_name="c", subcore_axis_name="s",
                                  num_cores=2, num_subcores=16)


def _make_sc_edge():
    scratch = [
        pltpu.VMEM((2, SEG, CH), jnp.int32),
        pltpu.VMEM((2, SEG, CH), jnp.float32),
        pltpu.VMEM((4, CH), jnp.int32),
        pltpu.VMEM((4, CH), jnp.int32),
        pltpu.VMEM((D,), jnp.float32),
        pltpu.VMEM((4, CH, D), jnp.float32),
        pltpu.VMEM_SHARED((NT, D), jnp.float32),
        pltpu.SemaphoreType.DMA,
        pltpu.SemaphoreType.DMA,
        pltpu.SemaphoreType.DMA,
        pltpu.SemaphoreType.DMA,
        pltpu.SemaphoreType.DMA,
        pltpu.SemaphoreType.DMA,
        pltpu.SemaphoreType.DMA,
        pltpu.SemaphoreType.DMA,
        pltpu.SemaphoreType.DMA,
    ]

    @functools.partial(pl.kernel,
                       out_type=jax.ShapeDtypeStruct((2, NT, D), jnp.float32),
                       mesh=_sc_mesh(), scratch_types=scratch)
    def sc_edge(hm_hbm, pk_hbm, ea_hbm, wm_hbm, z_hbm, agg_out,
                pk_sb, ea_sb, srcq, dstq, wm_v, rows4, aggS,
                g0, g1, g2, g3, s0, s1, s2, s3, sbsem):
        gsem = (g0, g1, g2, g3)
        ssem = (s0, s1, s2, s3)
        c = lax.axis_index("c")
        s = lax.axis_index("s")
        w = c * 16 + s

        pltpu.sync_copy(wm_hbm, wm_v)

        r0 = s * ROWS_PER_TILE
        pltpu.sync_copy(z_hbm.at[pl.ds(r0, ROWS_PER_TILE), :],
                        aggS.at[pl.ds(r0, ROWS_PER_TILE), :])
        plsc.subcore_barrier()

        def sb_load_async(q):
            sl = jnp.bitwise_and(q, 1)
            off = pl.multiple_of(q * SEG, SEG)
            pltpu.async_copy(pk_hbm.at[w, pl.ds(off, SEG), :],
                             pk_sb.at[sl], sbsem)
            pltpu.async_copy(ea_hbm.at[w, pl.ds(off, SEG), :],
                             ea_sb.at[sl], sbsem)

        def sb_wait():
            pltpu.make_async_copy(pk_hbm.at[0, pl.ds(0, SEG), :],
                                  pk_sb.at[0], sbsem).wait()
            pltpu.make_async_copy(pk_hbm.at[0, pl.ds(0, SEG), :],
                                  ea_sb.at[0], sbsem).wait()

        def unpack_idx(b, g):
            qsl = jnp.bitwise_and(lax.shift_right_logical(g, 4), 1)
            r = jnp.bitwise_and(g, SEG - 1)
            for u in range(CH // 16):
                sl = pl.ds(16 * u, 16)
                v = pk_sb[qsl, r, sl]
                srcq[b, sl] = jnp.bitwise_and(v, PK_MASK)
                dstq[b, sl] = lax.shift_right_logical(v, 14)

        def start_gather(b):
            pltpu.async_copy(hm_hbm.at[srcq.at[b]], rows4.at[b], gsem[b])

        def wait_gather(b):
            pltpu.make_async_copy(hm_hbm.at[pl.ds(0, CH), :], rows4.at[b],
                                  gsem[b]).wait()

        def start_scatter(b):
            pltpu.async_copy(rows4.at[b], aggS.at[dstq.at[b]], ssem[b],
                             add=True)

        def wait_scatter(b):
            pltpu.make_async_copy(hm_hbm.at[pl.ds(0, CH), :], rows4.at[b],
                                  ssem[b]).wait()

        dnums = lax.GatherDimensionNumbers(
            offset_dims=(), collapsed_slice_dims=(0,), start_index_map=(0,))

        def compute_chunk(b, g):
            qsl = jnp.bitwise_and(lax.shift_right_logical(g, 4), 1)
            r = jnp.bitwise_and(g, SEG - 1)
            wvs = tuple(wm_v[pl.ds(16 * j, 16)] for j in range(D // 16))

            def qbody(g16, wv):
                eav = ea_sb[qsl, r, pl.ds(16 * g16, 16)]

                def lbody(l, wv2):
                    eab = lax.gather(
                        eav, jnp.full((16, 1), l, jnp.int32), dnums,
                        slice_sizes=(1,),
                        mode=lax.GatherScatterMode.PROMISE_IN_BOUNDS)
                    e = 16 * g16 + l
                    for j in range(D // 16):
                        sl = pl.ds(j * 16, 16)
                        v = rows4[b, e, sl]
                        rows4[b, e, sl] = jnp.maximum(v + eab * wv2[j], 0.0)
                    return wv2
                return lax.fori_loop(0, 16, lbody, wv)
            lax.fori_loop(0, CH // 16, qbody, wvs)

        def step(b, g):
            wait_gather(b)
            compute_chunk(b, g)
            start_scatter(b)
            b2 = (b + 2) % 4
            wait_scatter(b2)
            unpack_idx(b2, g + 2)
            start_gather(b2)

        sb_load_async(jnp.int32(0))
        sb_wait()
        unpack_idx(0, jnp.int32(0))
        start_gather(0)
        unpack_idx(1, jnp.int32(1))
        start_gather(1)
        pltpu.sync_copy(z_hbm.at[pl.ds(0, CH), :], rows4.at[2])
        pltpu.sync_copy(z_hbm.at[pl.ds(0, CH), :], rows4.at[3])
        unpack_idx(2, jnp.int32(0))
        unpack_idx(3, jnp.int32(1))
        start_scatter(2)
        start_scatter(3)

        def seg_body(q, _):
            sb_load_async(q + 1)
            g0_ = q * SEG

            def quad(t2, _2):
                for i in range(4):
                    step(i, g0_ + 4 * t2 + i)
                return 0
            lax.fori_loop(0, (SEG - 4) // 4, quad, 0)
            sb_wait()
            for i in range(4):
                step(i, g0_ + SEG - 4 + i)
            return 0
        lax.fori_loop(0, NSEG, seg_body, 0)

        wait_gather(0)
        wait_gather(1)
        wait_scatter(2)
        wait_scatter(3)

        plsc.subcore_barrier()
        pltpu.sync_copy(aggS.at[pl.ds(r0, ROWS_PER_TILE), :],
                        agg_out.at[c, pl.ds(r0, ROWS_PER_TILE), :])

    return sc_edge


@functools.lru_cache(maxsize=None)
def _sc_kernels():
    return _make_sc_edge()


def kernel(x, edge_index, edge_attr, W_lin_0, b_lin_0, W_msg_0, b_msg_0,
           W_l_0, b_l_0, W_r_0, W_lin_1, b_lin_1, W_msg_1, b_msg_1, W_l_1,
           b_l_1, W_r_1, W_dec, b_dec, W_fin, b_fin):
    x_pad = jnp.pad(x, ((0, NT - N), (0, 0)))
    src = edge_index[0].astype(jnp.int32)
    dst = edge_index[1].astype(jnp.int32)
    ea = edge_attr[:, 0].astype(jnp.float32)
    npad = E_PAD - src.shape[0]
    padidx = (jnp.arange(npad, dtype=jnp.int32) % (NT - N)) + N
    src_p = jnp.concatenate([src, padidx])
    dst_p = jnp.concatenate([dst, padidx])
    pk = (dst_p * 16384 + src_p).reshape(NW, NCHUNK, CH)
    ea_p = jnp.concatenate([ea, jnp.zeros((npad,), jnp.float32)]
                           ).reshape(NW, NCHUNK, CH)
    zeros128 = jnp.zeros((NT, D), jnp.float32)

    _sc_edge = _sc_kernels()

    ones_ext = jnp.pad(jnp.ones((N, D), jnp.float32), ((0, NT - N), (0, 0)))
    wm_zero = jnp.zeros((D,), jnp.float32)
    cntp = _sc_edge(ones_ext, pk, ea_p, wm_zero, zeros128)

    hm0 = _tc_pre(x_pad, W_lin_0, b_lin_0, W_msg_0[:D], b_msg_0)
    aggp0 = _sc_edge(hm0, pk, ea_p, W_msg_0[D], zeros128)
    out0 = _tc_out(aggp0, cntp, x_pad, W_l_0, b_l_0, W_r_0)

    hm1 = _tc_pre(out0, W_lin_1, b_lin_1, W_msg_1[:D], b_msg_1)
    aggp1 = _sc_edge(hm1, pk, ea_p, W_msg_1[D], zeros128)
    out1 = _tc_out(aggp1, cntp, out0, W_l_1, b_l_1, W_r_1)

    logits = _tc_fin(out1, W_dec, b_dec, W_fin, b_fin)
    return logits[:N]


# speedup vs baseline: 4.3814x; 1.3572x over previous
import functools

import jax
import jax.numpy as jnp
from jax import lax
from jax.experimental import pallas as pl
from jax.experimental.pallas import tpu as pltpu
from jax.experimental.pallas import tpu_sc as plsc

N = 10000
D = 128
NT = 10240
NW = 32
CH = 64
SEG = 16
NSEG = 10
NCHUNK = SEG * NSEG
EPT = NCHUNK * CH
E_PAD = NW * EPT
ROWS_PER_TILE = NT // 16


BLK = 256
GRID = NT // BLK


def _pre_body(x_ref, wl_ref, bl_ref, wm_ref, bm_ref, o_ref):
    h = jnp.maximum(
        jnp.dot(x_ref[...], wl_ref[...], preferred_element_type=jnp.float32)
        + bl_ref[...], 0.0)
    v = jnp.dot(h, wm_ref[...], preferred_element_type=jnp.float32) + bm_ref[...]
    rows = pl.program_id(0) * BLK + lax.broadcasted_iota(jnp.int32, (BLK, 1), 0)
    o_ref[...] = jnp.where(rows < N, v, 0.0)


def _tc_pre(x_pad, Wl, bl, Wmh, bm):
    return pl.pallas_call(
        _pre_body,
        grid=(GRID,),
        in_specs=[
            pl.BlockSpec((BLK, D), lambda i: (i, 0)),
            pl.BlockSpec((D, D), lambda i: (0, 0)),
            pl.BlockSpec((1, D), lambda i: (0, 0)),
            pl.BlockSpec((D, D), lambda i: (0, 0)),
            pl.BlockSpec((1, D), lambda i: (0, 0)),
        ],
        out_specs=pl.BlockSpec((BLK, D), lambda i: (i, 0)),
        out_shape=jax.ShapeDtypeStruct((NT, D), jnp.float32),
    )(x_pad, Wl, bl.reshape(1, D), Wmh, bm.reshape(1, D))


def _out_body(a_ref, c_ref, x_ref, wll_ref, bll_ref, wr_ref, o_ref):
    cnt = (c_ref[0, :] + c_ref[1, :]).reshape(BLK, 1)
    agg = (a_ref[0] + a_ref[1]) * (1.0 / jnp.maximum(cnt, 1.0))
    v = (jnp.dot(agg, wll_ref[...], preferred_element_type=jnp.float32)
         + bll_ref[...]
         + jnp.dot(x_ref[...], wr_ref[...], preferred_element_type=jnp.float32))
    rows = pl.program_id(0) * BLK + lax.broadcasted_iota(jnp.int32, (BLK, 1), 0)
    o_ref[...] = jnp.where(rows < N, jnp.maximum(v, 0.0), 0.0)


def _tc_out(aggp, cntp, x_pad, Wll, bll, Wr):
    return pl.pallas_call(
        _out_body,
        grid=(GRID,),
        in_specs=[
            pl.BlockSpec((2, BLK, D), lambda i: (0, i, 0)),
            pl.BlockSpec((2, BLK), lambda i: (0, i)),
            pl.BlockSpec((BLK, D), lambda i: (i, 0)),
            pl.BlockSpec((D, D), lambda i: (0, 0)),
            pl.BlockSpec((1, D), lambda i: (0, 0)),
            pl.BlockSpec((D, D), lambda i: (0, 0)),
        ],
        out_specs=pl.BlockSpec((BLK, D), lambda i: (i, 0)),
        out_shape=jax.ShapeDtypeStruct((NT, D), jnp.float32),
    )(aggp, cntp, x_pad, Wll, bll.reshape(1, D), Wr)


def _fin_body(x_ref, wd_ref, bd_ref, wf_ref, bf_ref, o_ref):
    h = jnp.maximum(
        jnp.dot(x_ref[...], wd_ref[...], preferred_element_type=jnp.float32)
        + bd_ref[...], 0.0)
    o_ref[...] = (jnp.dot(h, wf_ref[...], preferred_element_type=jnp.float32)
                  + bf_ref[...])


def _tc_fin(h, Wd, bd, Wf, bf):
    return pl.pallas_call(
        _fin_body,
        grid=(GRID,),
        in_specs=[
            pl.BlockSpec((BLK, D), lambda i: (i, 0)),
            pl.BlockSpec((D, D), lambda i: (0, 0)),
            pl.BlockSpec((1, D), lambda i: (0, 0)),
            pl.BlockSpec((D, D), lambda i: (0, 0)),
            pl.BlockSpec((1, D), lambda i: (0, 0)),
        ],
        out_specs=pl.BlockSpec((BLK, D), lambda i: (i, 0)),
        out_shape=jax.ShapeDtypeStruct((NT, D), jnp.float32),
    )(h, Wd, bd.reshape(1, D), Wf, bf.reshape(1, D))


PK_MASK = 16383


def _sc_mesh():
    return plsc.VectorSubcoreMesh(core_axis_name="c", subcore_axis_name="s",
                                  num_cores=2, num_subcores=16)


def _make_sc_edge():
    scratch = [
        pltpu.VMEM((2, SEG, CH), jnp.int32),
        pltpu.VMEM((2, SEG, CH), jnp.float32),
        pltpu.VMEM((4, CH), jnp.int32),
        pltpu.VMEM((4, CH), jnp.int32),
        pltpu.VMEM((D,), jnp.float32),
        pltpu.VMEM((4, CH, D), jnp.float32),
        pltpu.VMEM_SHARED((NT, D), jnp.float32),
        pltpu.SemaphoreType.DMA,
        pltpu.SemaphoreType.DMA,
        pltpu.SemaphoreType.DMA,
        pltpu.SemaphoreType.DMA,
        pltpu.SemaphoreType.DMA,
        pltpu.SemaphoreType.DMA,
        pltpu.SemaphoreType.DMA,
        pltpu.SemaphoreType.DMA,
        pltpu.SemaphoreType.DMA,
    ]

    @functools.partial(pl.kernel,
                       out_type=jax.ShapeDtypeStruct((2, NT, D), jnp.float32),
                       mesh=_sc_mesh(), scratch_types=scratch)
    def sc_edge(hm_hbm, pk_hbm, ea_hbm, wm_hbm, z_hbm, agg_out,
                pk_sb, ea_sb, srcq, dstq, wm_v, rows4, aggS,
                g0, g1, g2, g3, s0, s1, s2, s3, sbsem):
        gsem = (g0, g1, g2, g3)
        ssem = (s0, s1, s2, s3)
        c = lax.axis_index("c")
        s = lax.axis_index("s")
        w = c * 16 + s

        pltpu.sync_copy(wm_hbm, wm_v)

        r0 = s * ROWS_PER_TILE
        pltpu.sync_copy(z_hbm.at[pl.ds(r0, ROWS_PER_TILE), :],
                        aggS.at[pl.ds(r0, ROWS_PER_TILE), :])
        plsc.subcore_barrier()

        def sb_load_async(q):
            sl = jnp.bitwise_and(q, 1)
            off = pl.multiple_of(q * SEG, SEG)
            pltpu.async_copy(pk_hbm.at[w, pl.ds(off, SEG), :],
                             pk_sb.at[sl], sbsem)
            pltpu.async_copy(ea_hbm.at[w, pl.ds(off, SEG), :],
                             ea_sb.at[sl], sbsem)

        def sb_wait():
            pltpu.make_async_copy(pk_hbm.at[0, pl.ds(0, SEG), :],
                                  pk_sb.at[0], sbsem).wait()
            pltpu.make_async_copy(pk_hbm.at[0, pl.ds(0, SEG), :],
                                  ea_sb.at[0], sbsem).wait()

        def unpack_idx(b, g):
            qsl = jnp.bitwise_and(lax.shift_right_logical(g, 4), 1)
            r = jnp.bitwise_and(g, SEG - 1)
            for u in range(CH // 16):
                sl = pl.ds(16 * u, 16)
                v = pk_sb[qsl, r, sl]
                srcq[b, sl] = jnp.bitwise_and(v, PK_MASK)
                dstq[b, sl] = lax.shift_right_logical(v, 14)

        def start_gather(b):
            pltpu.async_copy(hm_hbm.at[srcq.at[b]], rows4.at[b], gsem[b])

        def wait_gather(b):
            pltpu.make_async_copy(hm_hbm.at[pl.ds(0, CH), :], rows4.at[b],
                                  gsem[b]).wait()

        def start_scatter(b):
            pltpu.async_copy(rows4.at[b], aggS.at[dstq.at[b]], ssem[b],
                             add=True)

        def wait_scatter(b):
            pltpu.make_async_copy(hm_hbm.at[pl.ds(0, CH), :], rows4.at[b],
                                  ssem[b]).wait()

        dnums = lax.GatherDimensionNumbers(
            offset_dims=(), collapsed_slice_dims=(0,), start_index_map=(0,))

        def compute_chunk(b, g):
            qsl = jnp.bitwise_and(lax.shift_right_logical(g, 4), 1)
            r = jnp.bitwise_and(g, SEG - 1)
            wvs = tuple(wm_v[pl.ds(16 * j, 16)] for j in range(D // 16))

            def qbody(g16, wv):
                eav = ea_sb[qsl, r, pl.ds(16 * g16, 16)]

                def lbody(l, wv2):
                    eab = lax.gather(
                        eav, jnp.full((16, 1), l, jnp.int32), dnums,
                        slice_sizes=(1,),
                        mode=lax.GatherScatterMode.PROMISE_IN_BOUNDS)
                    e = 16 * g16 + l
                    for j in range(D // 16):
                        sl = pl.ds(j * 16, 16)
                        v = rows4[b, e, sl]
                        rows4[b, e, sl] = jnp.maximum(v + eab * wv2[j], 0.0)
                    return wv2
                return lax.fori_loop(0, 16, lbody, wv)
            lax.fori_loop(0, CH // 16, qbody, wvs)

        def step(b, g):
            wait_gather(b)
            compute_chunk(b, g)
            start_scatter(b)
            b2 = (b + 2) % 4
            wait_scatter(b2)
            unpack_idx(b2, g + 2)
            start_gather(b2)

        sb_load_async(jnp.int32(0))
        sb_wait()
        unpack_idx(0, jnp.int32(0))
        start_gather(0)
        unpack_idx(1, jnp.int32(1))
        start_gather(1)
        pltpu.sync_copy(z_hbm.at[pl.ds(0, CH), :], rows4.at[2])
        pltpu.sync_copy(z_hbm.at[pl.ds(0, CH), :], rows4.at[3])
        unpack_idx(2, jnp.int32(0))
        unpack_idx(3, jnp.int32(1))
        start_scatter(2)
        start_scatter(3)

        def seg_body(q, _):
            sb_load_async(q + 1)
            g0_ = q * SEG

            def quad(t2, _2):
                for i in range(4):
                    step(i, g0_ + 4 * t2 + i)
                return 0
            lax.fori_loop(0, (SEG - 4) // 4, quad, 0)
            sb_wait()
            for i in range(4):
                step(i, g0_ + SEG - 4 + i)
            return 0
        lax.fori_loop(0, NSEG, seg_body, 0)

        wait_gather(0)
        wait_gather(1)
        wait_scatter(2)
        wait_scatter(3)

        plsc.subcore_barrier()
        pltpu.sync_copy(aggS.at[pl.ds(r0, ROWS_PER_TILE), :],
                        agg_out.at[c, pl.ds(r0, ROWS_PER_TILE), :])

    return sc_edge


def _make_sc_cnt():
    scratch = [
        pltpu.VMEM((NCHUNK, CH), jnp.int32),
        pltpu.VMEM((CH,), jnp.int32),
        pltpu.VMEM((CH,), jnp.float32),
        pltpu.VMEM((ROWS_PER_TILE,), jnp.float32),
        pltpu.VMEM_SHARED((NT,), jnp.float32),
    ]

    @functools.partial(pl.kernel,
                       out_type=jax.ShapeDtypeStruct((2, NT), jnp.float32),
                       mesh=_sc_mesh(), scratch_types=scratch)
    def sc_cnt(pk_hbm, cnt_out, pk_v, dstq, ones_v, zb, cntS):
        c = lax.axis_index("c")
        s = lax.axis_index("s")
        w = c * 16 + s
        pltpu.sync_copy(pk_hbm.at[w], pk_v)

        def _fill(r, _):
            sl = pl.ds(16 * r, 16)
            ones_v[sl] = jnp.ones((16,), jnp.float32)
            return 0
        lax.fori_loop(0, CH // 16, _fill, 0)

        def _zfill(r, _):
            zb[pl.ds(16 * r, 16)] = jnp.zeros((16,), jnp.float32)
            return 0
        lax.fori_loop(0, ROWS_PER_TILE // 16, _zfill, 0)

        r0 = s * ROWS_PER_TILE
        pltpu.sync_copy(zb, cntS.at[pl.ds(r0, ROWS_PER_TILE)])
        plsc.subcore_barrier()

        def gbody(g, _):
            for u in range(CH // 16):
                sl = pl.ds(16 * u, 16)
                dstq[sl] = lax.shift_right_logical(pk_v[g, sl], 14)
            pltpu.sync_copy(ones_v, cntS.at[dstq], add=True)
            return 0
        lax.fori_loop(0, NCHUNK, gbody, 0)

        plsc.subcore_barrier()
        pltpu.sync_copy(cntS.at[pl.ds(r0, ROWS_PER_TILE)],
                        cnt_out.at[c, pl.ds(r0, ROWS_PER_TILE)])

    return sc_cnt


@functools.lru_cache(maxsize=None)
def _sc_kernels():
    return _make_sc_edge(), _make_sc_cnt()


def kernel(x, edge_index, edge_attr, W_lin_0, b_lin_0, W_msg_0, b_msg_0,
           W_l_0, b_l_0, W_r_0, W_lin_1, b_lin_1, W_msg_1, b_msg_1, W_l_1,
           b_l_1, W_r_1, W_dec, b_dec, W_fin, b_fin):
    x_pad = jnp.pad(x, ((0, NT - N), (0, 0)))
    src = edge_index[0].astype(jnp.int32)
    dst = edge_index[1].astype(jnp.int32)
    ea = edge_attr[:, 0].astype(jnp.float32)
    npad = E_PAD - src.shape[0]
    padidx = (jnp.arange(npad, dtype=jnp.int32) % (NT - N)) + N
    src_p = jnp.concatenate([src, padidx])
    dst_p = jnp.concatenate([dst, padidx])
    pk = (dst_p * 16384 + src_p).reshape(NW, NCHUNK, CH)
    ea_p = jnp.concatenate([ea, jnp.zeros((npad,), jnp.float32)]
                           ).reshape(NW, NCHUNK, CH)
    zeros128 = jnp.zeros((NT, D), jnp.float32)

    _sc_edge, _sc_cnt = _sc_kernels()

    cntp = _sc_cnt(pk)

    hm0 = _tc_pre(x_pad, W_lin_0, b_lin_0, W_msg_0[:D], b_msg_0)
    aggp0 = _sc_edge(hm0, pk, ea_p, W_msg_0[D], zeros128)
    out0 = _tc_out(aggp0, cntp, x_pad, W_l_0, b_l_0, W_r_0)

    hm1 = _tc_pre(out0, W_lin_1, b_lin_1, W_msg_1[:D], b_msg_1)
    aggp1 = _sc_edge(hm1, pk, ea_p, W_msg_1[D], zeros128)
    out1 = _tc_out(aggp1, cntp, out0, W_l_1, b_l_1, W_r_1)

    logits = _tc_fin(out1, W_dec, b_dec, W_fin, b_fin)
    return logits[:N]


# speedup vs baseline: 4.6298x; 1.0567x over previous
import functools

import jax
import jax.numpy as jnp
from jax import lax
from jax.experimental import pallas as pl
from jax.experimental.pallas import tpu as pltpu
from jax.experimental.pallas import tpu_sc as plsc

N = 10000
D = 128
NT = 10240
NW = 32
CH = 64
SEG = 16
NSEG = 10
NCHUNK = SEG * NSEG
EPT = NCHUNK * CH
E_PAD = NW * EPT
ROWS_PER_TILE = NT // 16


BLK = 256
GRID = NT // BLK


def _pre_body(x_ref, wl_ref, bl_ref, wm_ref, bm_ref, o_ref):
    h = jnp.maximum(
        jnp.dot(x_ref[...], wl_ref[...], preferred_element_type=jnp.float32)
        + bl_ref[...], 0.0)
    v = jnp.dot(h, wm_ref[...], preferred_element_type=jnp.float32) + bm_ref[...]
    rows = pl.program_id(0) * BLK + lax.broadcasted_iota(jnp.int32, (BLK, 1), 0)
    o_ref[...] = jnp.where(rows < N, v, 0.0)


def _tc_pre(x_pad, Wl, bl, Wmh, bm):
    return pl.pallas_call(
        _pre_body,
        grid=(GRID,),
        in_specs=[
            pl.BlockSpec((BLK, D), lambda i: (i, 0)),
            pl.BlockSpec((D, D), lambda i: (0, 0)),
            pl.BlockSpec((1, D), lambda i: (0, 0)),
            pl.BlockSpec((D, D), lambda i: (0, 0)),
            pl.BlockSpec((1, D), lambda i: (0, 0)),
        ],
        out_specs=pl.BlockSpec((BLK, D), lambda i: (i, 0)),
        out_shape=jax.ShapeDtypeStruct((NT, D), jnp.float32),
    )(x_pad, Wl, bl.reshape(1, D), Wmh, bm.reshape(1, D))


def _out_body(a_ref, c_ref, x_ref, wll_ref, bll_ref, wr_ref, o_ref):
    cnt = (c_ref[0, :] + c_ref[1, :]).reshape(BLK, 1)
    agg = (a_ref[0] + a_ref[1]) * (1.0 / jnp.maximum(cnt, 1.0))
    v = (jnp.dot(agg, wll_ref[...], preferred_element_type=jnp.float32)
         + bll_ref[...]
         + jnp.dot(x_ref[...], wr_ref[...], preferred_element_type=jnp.float32))
    rows = pl.program_id(0) * BLK + lax.broadcasted_iota(jnp.int32, (BLK, 1), 0)
    o_ref[...] = jnp.where(rows < N, jnp.maximum(v, 0.0), 0.0)


def _tc_out(aggp, cntp, x_pad, Wll, bll, Wr):
    return pl.pallas_call(
        _out_body,
        grid=(GRID,),
        in_specs=[
            pl.BlockSpec((2, BLK, D), lambda i: (0, i, 0)),
            pl.BlockSpec((2, BLK), lambda i: (0, i)),
            pl.BlockSpec((BLK, D), lambda i: (i, 0)),
            pl.BlockSpec((D, D), lambda i: (0, 0)),
            pl.BlockSpec((1, D), lambda i: (0, 0)),
            pl.BlockSpec((D, D), lambda i: (0, 0)),
        ],
        out_specs=pl.BlockSpec((BLK, D), lambda i: (i, 0)),
        out_shape=jax.ShapeDtypeStruct((NT, D), jnp.float32),
    )(aggp, cntp, x_pad, Wll, bll.reshape(1, D), Wr)


def _fin_body(x_ref, wd_ref, bd_ref, wf_ref, bf_ref, o_ref):
    h = jnp.maximum(
        jnp.dot(x_ref[...], wd_ref[...], preferred_element_type=jnp.float32)
        + bd_ref[...], 0.0)
    o_ref[...] = (jnp.dot(h, wf_ref[...], preferred_element_type=jnp.float32)
                  + bf_ref[...])


def _tc_fin(h, Wd, bd, Wf, bf):
    return pl.pallas_call(
        _fin_body,
        grid=(GRID,),
        in_specs=[
            pl.BlockSpec((BLK, D), lambda i: (i, 0)),
            pl.BlockSpec((D, D), lambda i: (0, 0)),
            pl.BlockSpec((1, D), lambda i: (0, 0)),
            pl.BlockSpec((D, D), lambda i: (0, 0)),
            pl.BlockSpec((1, D), lambda i: (0, 0)),
        ],
        out_specs=pl.BlockSpec((BLK, D), lambda i: (i, 0)),
        out_shape=jax.ShapeDtypeStruct((NT, D), jnp.float32),
    )(h, Wd, bd.reshape(1, D), Wf, bf.reshape(1, D))


PK_MASK = 16383


def _sc_mesh():
    return plsc.VectorSubcoreMesh(core_axis_name="c", subcore_axis_name="s",
                                  num_cores=2, num_subcores=16)


def _make_sc_edge():
    scratch = [
        pltpu.VMEM((2, SEG, CH), jnp.int32),
        pltpu.VMEM((2, SEG, CH), jnp.float32),
        pltpu.VMEM((4, CH), jnp.int32),
        pltpu.VMEM((4, CH), jnp.int32),
        pltpu.VMEM((D,), jnp.float32),
        pltpu.VMEM((4, CH, D), jnp.float32),
        pltpu.VMEM_SHARED((NT, D), jnp.float32),
        pltpu.SemaphoreType.DMA,
        pltpu.SemaphoreType.DMA,
        pltpu.SemaphoreType.DMA,
        pltpu.SemaphoreType.DMA,
        pltpu.SemaphoreType.DMA,
        pltpu.SemaphoreType.DMA,
        pltpu.SemaphoreType.DMA,
        pltpu.SemaphoreType.DMA,
        pltpu.SemaphoreType.DMA,
    ]

    @functools.partial(pl.kernel,
                       out_type=jax.ShapeDtypeStruct((2, NT, D), jnp.float32),
                       mesh=_sc_mesh(), scratch_types=scratch)
    def sc_edge(hm_hbm, pk_hbm, ea_hbm, wm_hbm, z_hbm, agg_out,
                pk_sb, ea_sb, srcq, dstq, wm_v, rows4, aggS,
                g0, g1, g2, g3, s0, s1, s2, s3, sbsem):
        gsem = (g0, g1, g2, g3)
        ssem = (s0, s1, s2, s3)
        c = lax.axis_index("c")
        s = lax.axis_index("s")
        w = c * 16 + s

        pltpu.sync_copy(wm_hbm, wm_v)

        r0 = s * ROWS_PER_TILE
        pltpu.sync_copy(z_hbm.at[pl.ds(r0, ROWS_PER_TILE), :],
                        aggS.at[pl.ds(r0, ROWS_PER_TILE), :])
        plsc.subcore_barrier()

        def sb_load_async(q):
            sl = jnp.bitwise_and(q, 1)
            off = pl.multiple_of(q * SEG, SEG)
            pltpu.async_copy(pk_hbm.at[w, pl.ds(off, SEG), :],
                             pk_sb.at[sl], sbsem)
            pltpu.async_copy(ea_hbm.at[w, pl.ds(off, SEG), :],
                             ea_sb.at[sl], sbsem)

        def sb_wait():
            pltpu.make_async_copy(pk_hbm.at[0, pl.ds(0, SEG), :],
                                  pk_sb.at[0], sbsem).wait()
            pltpu.make_async_copy(pk_hbm.at[0, pl.ds(0, SEG), :],
                                  ea_sb.at[0], sbsem).wait()

        def unpack_idx(b, g):
            qsl = jnp.bitwise_and(lax.shift_right_logical(g, 4), 1)
            r = jnp.bitwise_and(g, SEG - 1)
            for u in range(CH // 16):
                sl = pl.ds(16 * u, 16)
                v = pk_sb[qsl, r, sl]
                srcq[b, sl] = jnp.bitwise_and(v, PK_MASK)
                dstq[b, sl] = lax.shift_right_logical(v, 14)

        def start_gather(b):
            pltpu.async_copy(hm_hbm.at[srcq.at[b]], rows4.at[b], gsem[b])

        def wait_gather(b):
            pltpu.make_async_copy(hm_hbm.at[pl.ds(0, CH), :], rows4.at[b],
                                  gsem[b]).wait()

        def start_scatter(b):
            pltpu.async_copy(rows4.at[b], aggS.at[dstq.at[b]], ssem[b],
                             add=True)

        def wait_scatter(b):
            pltpu.make_async_copy(hm_hbm.at[pl.ds(0, CH), :], rows4.at[b],
                                  ssem[b]).wait()

        dnums = lax.GatherDimensionNumbers(
            offset_dims=(), collapsed_slice_dims=(0,), start_index_map=(0,))

        def compute_chunk(b, g):
            qsl = jnp.bitwise_and(lax.shift_right_logical(g, 4), 1)
            r = jnp.bitwise_and(g, SEG - 1)
            wvs = tuple(wm_v[pl.ds(16 * j, 16)] for j in range(D // 16))

            def qbody(g16, wv):
                eav = ea_sb[qsl, r, pl.ds(16 * g16, 16)]

                def lbody(p, wv2):
                    for dl in range(4):
                        l = 4 * p + dl
                        eab = lax.gather(
                            eav, jnp.full((16, 1), l, jnp.int32), dnums,
                            slice_sizes=(1,),
                            mode=lax.GatherScatterMode.PROMISE_IN_BOUNDS)
                        e = 16 * g16 + l
                        for j in range(D // 16):
                            sl = pl.ds(j * 16, 16)
                            v = rows4[b, e, sl]
                            rows4[b, e, sl] = jnp.maximum(
                                v + eab * wv2[j], 0.0)
                    return wv2
                return lax.fori_loop(0, 4, lbody, wv)
            lax.fori_loop(0, CH // 16, qbody, wvs)

        def step(b, g):
            wait_gather(b)
            compute_chunk(b, g)
            start_scatter(b)
            b2 = (b + 2) % 4
            wait_scatter(b2)
            unpack_idx(b2, g + 2)
            start_gather(b2)

        sb_load_async(jnp.int32(0))
        sb_wait()
        unpack_idx(0, jnp.int32(0))
        start_gather(0)
        unpack_idx(1, jnp.int32(1))
        start_gather(1)
        pltpu.sync_copy(z_hbm.at[pl.ds(0, CH), :], rows4.at[2])
        pltpu.sync_copy(z_hbm.at[pl.ds(0, CH), :], rows4.at[3])
        unpack_idx(2, jnp.int32(0))
        unpack_idx(3, jnp.int32(1))
        start_scatter(2)
        start_scatter(3)

        def seg_body(q, _):
            sb_load_async(q + 1)
            g0_ = q * SEG

            def quad(t2, _2):
                for i in range(4):
                    step(i, g0_ + 4 * t2 + i)
                return 0
            lax.fori_loop(0, (SEG - 4) // 4, quad, 0)
            sb_wait()
            for i in range(4):
                step(i, g0_ + SEG - 4 + i)
            return 0
        lax.fori_loop(0, NSEG, seg_body, 0)

        wait_gather(0)
        wait_gather(1)
        wait_scatter(2)
        wait_scatter(3)

        plsc.subcore_barrier()
        pltpu.sync_copy(aggS.at[pl.ds(r0, ROWS_PER_TILE), :],
                        agg_out.at[c, pl.ds(r0, ROWS_PER_TILE), :])

    return sc_edge


def _make_sc_cnt():
    scratch = [
        pltpu.VMEM((NCHUNK, CH), jnp.int32),
        pltpu.VMEM((CH,), jnp.int32),
        pltpu.VMEM((CH,), jnp.float32),
        pltpu.VMEM((ROWS_PER_TILE,), jnp.float32),
        pltpu.VMEM_SHARED((NT,), jnp.float32),
    ]

    @functools.partial(pl.kernel,
                       out_type=jax.ShapeDtypeStruct((2, NT), jnp.float32),
                       mesh=_sc_mesh(), scratch_types=scratch)
    def sc_cnt(pk_hbm, cnt_out, pk_v, dstq, ones_v, zb, cntS):
        c = lax.axis_index("c")
        s = lax.axis_index("s")
        w = c * 16 + s
        pltpu.sync_copy(pk_hbm.at[w], pk_v)

        def _fill(r, _):
            sl = pl.ds(16 * r, 16)
            ones_v[sl] = jnp.ones((16,), jnp.float32)
            return 0
        lax.fori_loop(0, CH // 16, _fill, 0)

        def _zfill(r, _):
            zb[pl.ds(16 * r, 16)] = jnp.zeros((16,), jnp.float32)
            return 0
        lax.fori_loop(0, ROWS_PER_TILE // 16, _zfill, 0)

        r0 = s * ROWS_PER_TILE
        pltpu.sync_copy(zb, cntS.at[pl.ds(r0, ROWS_PER_TILE)])
        plsc.subcore_barrier()

        def gbody(g, _):
            for u in range(CH // 16):
                sl = pl.ds(16 * u, 16)
                dstq[sl] = lax.shift_right_logical(pk_v[g, sl], 14)
            pltpu.sync_copy(ones_v, cntS.at[dstq], add=True)
            return 0
        lax.fori_loop(0, NCHUNK, gbody, 0)

        plsc.subcore_barrier()
        pltpu.sync_copy(cntS.at[pl.ds(r0, ROWS_PER_TILE)],
                        cnt_out.at[c, pl.ds(r0, ROWS_PER_TILE)])

    return sc_cnt


@functools.lru_cache(maxsize=None)
def _sc_kernels():
    return _make_sc_edge(), _make_sc_cnt()


def kernel(x, edge_index, edge_attr, W_lin_0, b_lin_0, W_msg_0, b_msg_0,
           W_l_0, b_l_0, W_r_0, W_lin_1, b_lin_1, W_msg_1, b_msg_1, W_l_1,
           b_l_1, W_r_1, W_dec, b_dec, W_fin, b_fin):
    x_pad = jnp.pad(x, ((0, NT - N), (0, 0)))
    src = edge_index[0].astype(jnp.int32)
    dst = edge_index[1].astype(jnp.int32)
    ea = edge_attr[:, 0].astype(jnp.float32)
    npad = E_PAD - src.shape[0]
    padidx = (jnp.arange(npad, dtype=jnp.int32) % (NT - N)) + N
    src_p = jnp.concatenate([src, padidx])
    dst_p = jnp.concatenate([dst, padidx])
    pk = (dst_p * 16384 + src_p).reshape(NW, NCHUNK, CH)
    ea_p = jnp.concatenate([ea, jnp.zeros((npad,), jnp.float32)]
                           ).reshape(NW, NCHUNK, CH)
    zeros128 = jnp.zeros((NT, D), jnp.float32)

    _sc_edge, _sc_cnt = _sc_kernels()

    cntp = _sc_cnt(pk)

    hm0 = _tc_pre(x_pad, W_lin_0, b_lin_0, W_msg_0[:D], b_msg_0)
    aggp0 = _sc_edge(hm0, pk, ea_p, W_msg_0[D], zeros128)
    out0 = _tc_out(aggp0, cntp, x_pad, W_l_0, b_l_0, W_r_0)

    hm1 = _tc_pre(out0, W_lin_1, b_lin_1, W_msg_1[:D], b_msg_1)
    aggp1 = _sc_edge(hm1, pk, ea_p, W_msg_1[D], zeros128)
    out1 = _tc_out(aggp1, cntp, out0, W_l_1, b_l_1, W_r_1)

    logits = _tc_fin(out1, W_dec, b_dec, W_fin, b_fin)
    return logits[:N]
